# baseline (device time: 16973 ns/iter reference)
import jax
import jax.numpy as jnp
from jax import lax
from jax.experimental import pallas as pl
from jax.experimental.pallas import tpu as pltpu

B, H, D, BS = 16, 16, 64, 16
P_LOC = 128
NSLOTS = 128
K_LOC = P_LOC * BS
HD = H * D
HB = H * B
NEG = -1e30


def kernel(Q, K, V, bt, lens):
    KT = jnp.transpose(K, (1, 2, 3, 0))
    VT = jnp.transpose(V, (1, 2, 3, 0))

    def body(q_ref, kt_ref, vt_ref, bt_ref, lens_ref, out_ref,
             kt_vmem, vt_vmem, o_send, o_recv, st_send, st_recv,
             send_sems, recv_sems, local_sems):
        my_x = lax.axis_index("x")
        my_y = lax.axis_index("y")
        my_z = lax.axis_index("z")
        partner = (my_x, 1 - my_y, my_z)

        CH = 4
        NCH = BS // CH
        k_copies, v_copies = [], []
        for c in range(NCH):
            sl = pl.ds(c * CH, CH)
            k_copies.append(pltpu.make_async_copy(
                kt_ref.at[sl], kt_vmem.at[sl], local_sems.at[0, c]))
            v_copies.append(pltpu.make_async_copy(
                vt_ref.at[sl], vt_vmem.at[sl], local_sems.at[1, c]))
        for c in range(NCH):
            k_copies[c].start()
        for c in range(NCH):
            v_copies[c].start()

        with jax.named_scope("barrier"):
            barrier_sem = pltpu.get_barrier_semaphore()
            pl.semaphore_signal(barrier_sem, inc=1, device_id=partner,
                                device_id_type=pl.DeviceIdType.MESH)
            pl.semaphore_wait(barrier_sem, 1)

        with jax.named_scope("multiplicity"):
            btT = jnp.transpose(bt_ref[:])
            lens_row = jnp.reshape(lens_ref[:], (1, B))
            validT = (lax.broadcasted_iota(jnp.int32, (NSLOTS, B), 0)
                      < lens_row)
            btT_m = jnp.where(validT, btT, -1)
            page_row = (my_y * P_LOC
                        + lax.broadcasted_iota(jnp.int32, (1, P_LOC), 1))

        def c_row(b):
            eq = (btT_m[:, b:b + 1] == page_row).astype(jnp.float32)
            return jnp.sum(eq, axis=0, keepdims=True)

        with jax.named_scope("qbig"):
            q2 = jnp.reshape(q_ref[:], (B, HD))
            q3 = jnp.transpose(q2)
            q_rep = jnp.tile(q3, (1, H))
            row_h = lax.broadcasted_iota(jnp.int32, (HD, HB), 0) // D
            col_h = lax.broadcasted_iota(jnp.int32, (HD, HB), 1) // B
            q_big = jnp.where(row_h == col_h, q_rep, 0.0).astype(jnp.bfloat16)

        HG = H // 2
        rdma_o = [
            pltpu.make_async_remote_copy(
                src_ref=o_send.at[pl.ds(g * HG, HG)],
                dst_ref=o_recv.at[pl.ds(g * HG, HG)],
                send_sem=send_sems.at[g], recv_sem=recv_sems.at[g],
                device_id=partner, device_id_type=pl.DeviceIdType.MESH)
            for g in range(2)
        ]
        rdma_st = pltpu.make_async_remote_copy(
            src_ref=st_send, dst_ref=st_recv,
            send_sem=send_sems.at[2], recv_sem=recv_sems.at[2],
            device_id=partner, device_id_type=pl.DeviceIdType.MESH)

        with jax.named_scope("qk"):
            s_parts = []
            c_rows = []
            for bs in range(BS):
                if bs % CH == 0:
                    k_copies[bs // CH].wait()
                k_slab = jnp.reshape(kt_vmem[bs], (HD, P_LOC))
                s_parts.append(lax.dot_general(
                    q_big, k_slab.astype(jnp.bfloat16),
                    (((0,), (0,)), ((), ())),
                    preferred_element_type=jnp.float32))
                c_rows.append(c_row(bs))
            s = jnp.concatenate(s_parts, axis=1)
            c_page = jnp.concatenate(c_rows, axis=0)
            c_keys = jnp.tile(c_page, (1, BS))
            c_big = jnp.tile(c_keys, (H, 1))
            has = c_big > 0.0

        with jax.named_scope("softmax"):
            s = jnp.where(has, s * (D ** -0.5), NEG)
            m = jnp.max(s, axis=1, keepdims=True)
            p = jnp.exp(s - m) * c_big
            l = jnp.sum(p, axis=1, keepdims=True)
            p16 = p.astype(jnp.bfloat16)
            st_send[0] = m
            st_send[1] = l
            rdma_st.start()

        with jax.named_scope("pv"):
            for c in range(NCH):
                v_copies[c].wait()
            v16 = [jnp.reshape(vt_vmem[bs], (HD, P_LOC)).astype(jnp.bfloat16)
                   for bs in range(BS)]
            for g in range(2):
                rows = slice(g * HG * B, (g + 1) * HG * B)
                acc = None
                for bs in range(BS):
                    t = lax.dot_general(
                        p16[rows, bs * P_LOC:(bs + 1) * P_LOC],
                        v16[bs],
                        (((1,), (1,)), ((), ())),
                        preferred_element_type=jnp.float32)
                    acc = t if acc is None else acc + t
                for h in range(g * HG, (g + 1) * HG):
                    o_send[h] = acc[(h - g * HG) * B:(h - g * HG + 1) * B,
                                    h * D:(h + 1) * D]
                rdma_o[g].start()

        with jax.named_scope("exchange"):
            rdma_o[0].wait()
            rdma_o[1].wait()
            rdma_st.wait()

        with jax.named_scope("combine"):
            m_a, l_a = st_send[0], st_send[1]
            m_b, l_b = st_recv[0], st_recv[1]
            m_f = jnp.maximum(m_a, m_b)
            w_a = jnp.exp(m_a - m_f)
            w_b = jnp.exp(m_b - m_f)
            l_f = w_a * l_a + w_b * l_b
            for h in range(H):
                sl = slice(h * B, (h + 1) * B)
                o_c = (w_a[sl] * o_send[h]
                       + w_b[sl] * o_recv[h]) / l_f[sl]
                out_ref[:, 0, h, :] = o_c

    return pl.pallas_call(
        body,
        out_shape=jax.ShapeDtypeStruct((B, 1, H, D), jnp.float32),
        in_specs=[
            pl.BlockSpec(memory_space=pltpu.VMEM),
            pl.BlockSpec(memory_space=pltpu.MemorySpace.HBM),
            pl.BlockSpec(memory_space=pltpu.MemorySpace.HBM),
            pl.BlockSpec(memory_space=pltpu.VMEM),
            pl.BlockSpec(memory_space=pltpu.VMEM),
        ],
        out_specs=pl.BlockSpec(memory_space=pltpu.VMEM),
        scratch_shapes=[
            pltpu.VMEM((BS, H, D, P_LOC), jnp.float32),
            pltpu.VMEM((BS, H, D, P_LOC), jnp.float32),
            pltpu.VMEM((H, B, D), jnp.float32),
            pltpu.VMEM((H, B, D), jnp.float32),
            pltpu.VMEM((2, HB, 1), jnp.float32),
            pltpu.VMEM((2, HB, 1), jnp.float32),
            pltpu.SemaphoreType.DMA((3,)),
            pltpu.SemaphoreType.DMA((3,)),
            pltpu.SemaphoreType.DMA((2, 4)),
        ],
        compiler_params=pltpu.CompilerParams(
            collective_id=0,
            vmem_limit_bytes=100 * 1024 * 1024,
        ),
    )(Q, KT, VT, bt, lens)


# device time: 16885 ns/iter; 1.0052x vs baseline; 1.0052x over previous
import jax
import jax.numpy as jnp
from jax import lax
from jax.experimental import pallas as pl
from jax.experimental.pallas import tpu as pltpu

B, H, D, BS = 16, 16, 64, 16
P_LOC = 128
NSLOTS = 128
K_LOC = P_LOC * BS
HD = H * D
HB = H * B
NEG = -1e30


def kernel(Q, K, V, bt, lens):
    KT = jnp.transpose(K, (1, 2, 3, 0))
    VT = jnp.transpose(V, (1, 2, 3, 0))

    def body(q_ref, kt_ref, vt_ref, bt_ref, lens_ref, out_ref,
             kt_vmem, vt_vmem, out_vmem, o_send, o_recv, st_send, st_recv,
             send_sems, recv_sems, local_sems):
        my_x = lax.axis_index("x")
        my_y = lax.axis_index("y")
        my_z = lax.axis_index("z")
        partner = (my_x, 1 - my_y, my_z)

        CH = 4
        NCH = BS // CH
        k_copies, v_copies = [], []
        for c in range(NCH):
            sl = pl.ds(c * CH, CH)
            k_copies.append(pltpu.make_async_copy(
                kt_ref.at[sl], kt_vmem.at[sl], local_sems.at[0, c]))
            v_copies.append(pltpu.make_async_copy(
                vt_ref.at[sl], vt_vmem.at[sl], local_sems.at[1, c]))
        for c in range(NCH):
            k_copies[c].start()
        for c in range(NCH):
            v_copies[c].start()

        with jax.named_scope("barrier"):
            barrier_sem = pltpu.get_barrier_semaphore()
            pl.semaphore_signal(barrier_sem, inc=1, device_id=partner,
                                device_id_type=pl.DeviceIdType.MESH)
            pl.semaphore_wait(barrier_sem, 1)

        with jax.named_scope("multiplicity"):
            btT = jnp.transpose(bt_ref[:])
            lens_row = jnp.reshape(lens_ref[:], (1, B))
            validT = (lax.broadcasted_iota(jnp.int32, (NSLOTS, B), 0)
                      < lens_row)
            btT_m = jnp.where(validT, btT, -1)
            page_row = (my_y * P_LOC
                        + lax.broadcasted_iota(jnp.int32, (1, P_LOC), 1))

        def c_row(b):
            eq = (btT_m[:, b:b + 1] == page_row).astype(jnp.float32)
            return jnp.sum(eq, axis=0, keepdims=True)

        with jax.named_scope("qbig"):
            q2 = jnp.reshape(q_ref[:], (B, HD))
            q3 = jnp.transpose(q2)
            q_rep = jnp.tile(q3, (1, H))
            row_h = lax.broadcasted_iota(jnp.int32, (HD, HB), 0) // D
            col_h = lax.broadcasted_iota(jnp.int32, (HD, HB), 1) // B
            q_big = jnp.where(row_h == col_h, q_rep, 0.0).astype(jnp.bfloat16)

        HG = H // 2
        rdma_o = [
            pltpu.make_async_remote_copy(
                src_ref=o_send.at[pl.ds(g * HG, HG)],
                dst_ref=o_recv.at[pl.ds(g * HG, HG)],
                send_sem=send_sems.at[g], recv_sem=recv_sems.at[g],
                device_id=partner, device_id_type=pl.DeviceIdType.MESH)
            for g in range(2)
        ]
        rdma_st = pltpu.make_async_remote_copy(
            src_ref=st_send, dst_ref=st_recv,
            send_sem=send_sems.at[2], recv_sem=recv_sems.at[2],
            device_id=partner, device_id_type=pl.DeviceIdType.MESH)

        with jax.named_scope("qk"):
            s_parts = []
            c_rows = []
            for bs in range(BS):
                if bs % CH == 0:
                    k_copies[bs // CH].wait()
                k_slab = jnp.reshape(kt_vmem[bs], (HD, P_LOC))
                s_parts.append(lax.dot_general(
                    q_big, k_slab.astype(jnp.bfloat16),
                    (((0,), (0,)), ((), ())),
                    preferred_element_type=jnp.float32))
                c_rows.append(c_row(bs))
            s = jnp.concatenate(s_parts, axis=1)
            c_page = jnp.concatenate(c_rows, axis=0)
            c_keys = jnp.tile(c_page, (1, BS))
            c_big = jnp.tile(c_keys, (H, 1))
            has = c_big > 0.0

        with jax.named_scope("softmax"):
            s = jnp.where(has, s * (D ** -0.5), NEG)
            m = jnp.max(s, axis=1, keepdims=True)
            p = jnp.exp(s - m) * c_big
            l = jnp.sum(p, axis=1, keepdims=True)
            p16 = p.astype(jnp.bfloat16)
            st_send[0] = m
            st_send[1] = l
            rdma_st.start()

        with jax.named_scope("pv"):
            for c in range(NCH):
                v_copies[c].wait()
            v16 = [jnp.reshape(vt_vmem[bs], (HD, P_LOC)).astype(jnp.bfloat16)
                   for bs in range(BS)]
            for g in range(2):
                rows = slice(g * HG * B, (g + 1) * HG * B)
                acc = None
                for bs in range(BS):
                    t = lax.dot_general(
                        p16[rows, bs * P_LOC:(bs + 1) * P_LOC],
                        v16[bs],
                        (((1,), (1,)), ((), ())),
                        preferred_element_type=jnp.float32)
                    acc = t if acc is None else acc + t
                for h in range(g * HG, (g + 1) * HG):
                    o_send[h] = acc[(h - g * HG) * B:(h - g * HG + 1) * B,
                                    h * D:(h + 1) * D]
                rdma_o[g].start()

        with jax.named_scope("exchange"):
            rdma_o[0].wait()
            rdma_o[1].wait()
            rdma_st.wait()

        with jax.named_scope("combine"):
            m_a, l_a = st_send[0], st_send[1]
            m_b, l_b = st_recv[0], st_recv[1]
            m_f = jnp.maximum(m_a, m_b)
            w_a = jnp.exp(m_a - m_f)
            w_b = jnp.exp(m_b - m_f)
            l_f = w_a * l_a + w_b * l_b
            for h in range(H):
                sl = slice(h * B, (h + 1) * B)
                o_c = (w_a[sl] * o_send[h]
                       + w_b[sl] * o_recv[h]) / l_f[sl]
                out_vmem[:, 0, h, :] = o_c
            copy_out = pltpu.make_async_copy(out_vmem, out_ref,
                                             local_sems.at[0, 0])
            copy_out.start()
            copy_out.wait()

    return pl.pallas_call(
        body,
        out_shape=jax.ShapeDtypeStruct((B, 1, H, D), jnp.float32),
        in_specs=[
            pl.BlockSpec(memory_space=pltpu.VMEM),
            pl.BlockSpec(memory_space=pltpu.MemorySpace.HBM),
            pl.BlockSpec(memory_space=pltpu.MemorySpace.HBM),
            pl.BlockSpec(memory_space=pltpu.VMEM),
            pl.BlockSpec(memory_space=pltpu.VMEM),
        ],
        out_specs=pl.BlockSpec(memory_space=pl.MemorySpace.ANY),
        scratch_shapes=[
            pltpu.VMEM((BS, H, D, P_LOC), jnp.float32),
            pltpu.VMEM((BS, H, D, P_LOC), jnp.float32),
            pltpu.VMEM((B, 1, H, D), jnp.float32),
            pltpu.VMEM((H, B, D), jnp.float32),
            pltpu.VMEM((H, B, D), jnp.float32),
            pltpu.VMEM((2, HB, 1), jnp.float32),
            pltpu.VMEM((2, HB, 1), jnp.float32),
            pltpu.SemaphoreType.DMA((3,)),
            pltpu.SemaphoreType.DMA((3,)),
            pltpu.SemaphoreType.DMA((2, 4)),
        ],
        compiler_params=pltpu.CompilerParams(
            collective_id=0,
            vmem_limit_bytes=100 * 1024 * 1024,
        ),
    )(Q, KT, VT, bt, lens)
